# baseline (speedup 1.0000x reference)
import functools

import jax
import jax.numpy as jnp
from jax import lax
from jax.experimental import pallas as pl
from jax.experimental.pallas import tpu as pltpu
from jax.experimental.pallas import tpu_sc as plsc

_N = 10000
_E = 320000
_H = 128
_G = 64

_NC = 2
_NS = 16
_NW = _NC * _NS

_C = 64
_EPW = 10240
_EP = _EPW * _NW
_NCH = _EPW // _C
_ACC_N = _N + 8
_ZR = 632
_ZMAX = _N - _ZR

_BN = 2000
_PREC = jax.lax.Precision.DEFAULT

_mesh = plsc.VectorSubcoreMesh(core_axis_name="c", subcore_axis_name="s")


_NSLOT = 2


@functools.partial(
    pl.kernel,
    mesh=_mesh,
    out_type=jax.ShapeDtypeStruct((_NC, _N, _H), jnp.float32),
    scratch_types=[
        pltpu.VMEM((_EPW,), jnp.int32),
        pltpu.VMEM((_NCH, _C), jnp.int32),
        pltpu.VMEM((_NSLOT, _C, _H), jnp.float32),
        pltpu.VMEM_SHARED((_ACC_N, _H), jnp.float32),
    ] + [pltpu.SemaphoreType.DMA] * (2 * _NSLOT),
)
def _seg_sum(h_hbm, src_hbm, dst_hbm, z_hbm, out_hbm, srcv, dstv, rows, acc,
             *sems):
  gsem = sems[:_NSLOT]
  ssem = sems[_NSLOT:]
  cid = lax.axis_index("c")
  sid = lax.axis_index("s")
  wid = sid * _NC + cid

  def start_gather(ci, s):
    pltpu.async_copy(h_hbm.at[srcv.at[pl.ds(ci * _C, _C)]], rows.at[s],
                     gsem[s])

  def wait_gather(ci, s):
    pltpu.make_async_copy(h_hbm.at[srcv.at[pl.ds(ci * _C, _C)]], rows.at[s],
                          gsem[s]).wait()

  def start_scatter(ci, s):
    pltpu.async_copy(rows.at[s], acc.at[dstv.at[ci]], ssem[s], add=True)

  def wait_scatter(s):
    pltpu.make_async_copy(rows.at[s], acc.at[dstv.at[0]], ssem[s]).wait()

  pltpu.sync_copy(src_hbm.at[pl.ds(wid * _EPW, _EPW)], srcv)
  pltpu.sync_copy(dst_hbm.at[pl.ds(wid * _NCH, _NCH)], dstv)
  for s in range(_NSLOT):
    start_gather(s, s)
  zoff = jnp.minimum(sid * _ZR, _ZMAX)
  pltpu.sync_copy(z_hbm, acc.at[pl.ds(zoff, _ZR)])

  @pl.when(sid == _NS - 1)
  def _():
    pltpu.sync_copy(z_hbm.at[pl.ds(0, 8)], acc.at[pl.ds(_N, 8)])

  plsc.subcore_barrier()

  @pl.loop(0, _NCH, step=_NSLOT)
  def _(i):
    for s in range(_NSLOT):
      wait_gather(i + s, s)
      start_scatter(i + s, s)
    for s in range(_NSLOT):
      j = i + _NSLOT + s

      @pl.when(j < _NCH)
      def _():
        wait_scatter(s)
        start_gather(j, s)

  for s in range(_NSLOT):
    wait_scatter(s)

  plsc.subcore_barrier()
  pltpu.sync_copy(acc.at[pl.ds(zoff, _ZR)],
                  out_hbm.at[cid, pl.ds(zoff, _ZR)])


def _tc_pre(xp, wrel, wroot, b):
  n, d_in = xp.shape
  grid = (n // _BN,)

  def body(x_ref, wrel_ref, wroot_ref, b_ref, y_ref, r_ref):
    xb = x_ref[...]
    y_ref[...] = jnp.dot(xb, wrel_ref[...],
                         preferred_element_type=jnp.float32, precision=_PREC)
    r_ref[...] = jnp.dot(xb, wroot_ref[...],
                         preferred_element_type=jnp.float32,
                         precision=_PREC) + b_ref[...]

  return pl.pallas_call(
      body,
      grid=grid,
      in_specs=[
          pl.BlockSpec((_BN, d_in), lambda i: (i, 0)),
          pl.BlockSpec((d_in, _H), lambda i: (0, 0)),
          pl.BlockSpec((d_in, _H), lambda i: (0, 0)),
          pl.BlockSpec((1, _H), lambda i: (0, 0)),
      ],
      out_specs=[
          pl.BlockSpec((_BN, _H), lambda i: (i, 0)),
          pl.BlockSpec((_BN, _H), lambda i: (i, 0)),
      ],
      out_shape=[
          jax.ShapeDtypeStruct((n, _H), jnp.float32),
          jax.ShapeDtypeStruct((n, _H), jnp.float32),
      ],
  )(xp, wrel, wroot, b)


def _tc_mid(p, r, wrel, wroot, b):
  n = r.shape[0]
  grid = (n // _BN,)

  def body(p_ref, r_ref, wrel_ref, wroot_ref, b_ref, y_ref, rn_ref):
    h = jnp.maximum(p_ref[0] + p_ref[1] + r_ref[...], 0.0)
    y_ref[...] = jnp.dot(h, wrel_ref[...],
                         preferred_element_type=jnp.float32, precision=_PREC)
    rn_ref[...] = jnp.dot(h, wroot_ref[...],
                          preferred_element_type=jnp.float32,
                          precision=_PREC) + b_ref[...]

  return pl.pallas_call(
      body,
      grid=grid,
      in_specs=[
          pl.BlockSpec((_NC, _BN, _H), lambda i: (0, i, 0)),
          pl.BlockSpec((_BN, _H), lambda i: (i, 0)),
          pl.BlockSpec((_H, _H), lambda i: (0, 0)),
          pl.BlockSpec((_H, _H), lambda i: (0, 0)),
          pl.BlockSpec((1, _H), lambda i: (0, 0)),
      ],
      out_specs=[
          pl.BlockSpec((_BN, _H), lambda i: (i, 0)),
          pl.BlockSpec((_BN, _H), lambda i: (i, 0)),
      ],
      out_shape=[
          jax.ShapeDtypeStruct((n, _H), jnp.float32),
          jax.ShapeDtypeStruct((n, _H), jnp.float32),
      ],
  )(p, r, wrel, wroot, b)


def _tc_final(p, r, batch3d, wimp, bimp, wsta, wstb, bst):
  n = r.shape[0]
  grid_n = n // _BN

  def body(p_ref, r_ref, batch_ref, wimp_ref, bimp_ref, wsta_ref, wstb_ref,
           bst_ref, imp_ref, st_ref, psum, pcnt):
    i = pl.program_id(0)

    @pl.when(i == 0)
    def _():
      psum[...] = jnp.zeros_like(psum)
      pcnt[...] = jnp.zeros_like(pcnt)

    h3 = p_ref[0] + p_ref[1] + r_ref[...]
    bvec = batch_ref[0]
    sel = (lax.broadcasted_iota(jnp.int32, (_G, _BN), 0) == bvec)
    sel = sel.astype(jnp.float32)
    psum[...] += jnp.dot(sel, h3, preferred_element_type=jnp.float32,
                         precision=_PREC)
    pcnt[...] += jnp.sum(sel, axis=1, keepdims=True)

    @pl.when(i == grid_n - 1)
    def _():
      pooled = psum[...] / jnp.maximum(pcnt[...], 1.0)
      imp = jnp.dot(pooled, wimp_ref[...], preferred_element_type=jnp.float32,
                    precision=_PREC) + bimp_ref[...]
      st = jnp.dot(pooled, wsta_ref[...], preferred_element_type=jnp.float32,
                   precision=_PREC)
      st += jnp.dot(imp, wstb_ref[...], preferred_element_type=jnp.float32,
                    precision=_PREC)
      st += bst_ref[...]
      imp_ref[...] = imp
      st_ref[...] = st

  return pl.pallas_call(
      body,
      grid=(grid_n,),
      in_specs=[
          pl.BlockSpec((_NC, _BN, _H), lambda i: (0, i, 0)),
          pl.BlockSpec((_BN, _H), lambda i: (i, 0)),
          pl.BlockSpec((1, 1, _BN), lambda i: (i, 0, 0)),
          pl.BlockSpec((_H, 3), lambda i: (0, 0)),
          pl.BlockSpec((1, 3), lambda i: (0, 0)),
          pl.BlockSpec((_H, 3), lambda i: (0, 0)),
          pl.BlockSpec((3, 3), lambda i: (0, 0)),
          pl.BlockSpec((1, 3), lambda i: (0, 0)),
      ],
      out_specs=[
          pl.BlockSpec((_G, 3), lambda i: (0, 0)),
          pl.BlockSpec((_G, 3), lambda i: (0, 0)),
      ],
      out_shape=[
          jax.ShapeDtypeStruct((_G, 3), jnp.float32),
          jax.ShapeDtypeStruct((_G, 3), jnp.float32),
      ],
      scratch_shapes=[
          pltpu.VMEM((_G, _H), jnp.float32),
          pltpu.VMEM((_G, 1), jnp.float32),
      ],
  )(p, r, batch3d, wimp, bimp, wsta, wstb, bst)


def kernel(x, edge_index, batch, Wrel1, Wroot1, b1, Wrel2, Wroot2, b2,
           Wrel3, Wroot3, b3, Wimp, bimp, Wst, bst):
  pad = _EP - _E
  src_flat = jnp.concatenate(
      [edge_index[0], jnp.zeros((pad,), jnp.int32)])
  dst2d = jnp.concatenate(
      [edge_index[1], jnp.full((pad,), _N, jnp.int32)]).reshape(-1, _C)
  xp = jnp.pad(x, ((0, 0), (0, 8 - x.shape[1])))
  wrel1p = jnp.pad(Wrel1, ((0, 8 - Wrel1.shape[0]), (0, 0)))
  wroot1p = jnp.pad(Wroot1, ((0, 8 - Wroot1.shape[0]), (0, 0)))
  zeros = jnp.zeros((_ZR, _H), jnp.float32)
  batch3d = batch.reshape(_N // _BN, 1, _BN)

  y1, r1 = _tc_pre(xp, wrel1p, wroot1p, b1.reshape(1, _H))
  p1 = _seg_sum(y1, src_flat, dst2d, zeros)
  y2, r2 = _tc_mid(p1, r1, Wrel2, Wroot2, b2.reshape(1, _H))
  p2 = _seg_sum(y2, src_flat, dst2d, zeros)
  y3, r3 = _tc_mid(p2, r2, Wrel3, Wroot3, b3.reshape(1, _H))
  p3 = _seg_sum(y3, src_flat, dst2d, zeros)
  imp, st = _tc_final(p3, r3, batch3d, Wimp, bimp.reshape(1, 3),
                      Wst[:_H], Wst[_H:], bst.reshape(1, 3))
  return (imp, st)


# speedup vs baseline: 2.7871x; 2.7871x over previous
import functools

import jax
import jax.numpy as jnp
from jax import lax
from jax.experimental import pallas as pl
from jax.experimental.pallas import tpu as pltpu
from jax.experimental.pallas import tpu_sc as plsc

_N = 10000
_E = 320000
_H = 128
_G = 64

_NC = 2
_NS = 16
_NW = _NC * _NS

_C = 80
_EPW = _E // _NW
_NCH = _EPW // _C
_ZR = 632
_ZMAX = _N - _ZR

_BN = 2000
_PREC = jax.lax.Precision.DEFAULT

_mesh = plsc.VectorSubcoreMesh(core_axis_name="c", subcore_axis_name="s")


_NSLOT = 2


@functools.partial(
    pl.kernel,
    mesh=_mesh,
    out_type=jax.ShapeDtypeStruct((_NC, _N, _H), jnp.float32),
    scratch_types=[
        pltpu.VMEM((_EPW,), jnp.int32),
        pltpu.VMEM((_NCH, _C), jnp.int32),
        pltpu.VMEM((_NSLOT, _C, _H), jnp.float32),
        pltpu.VMEM_SHARED((_N, _H), jnp.float32),
    ] + [pltpu.SemaphoreType.DMA] * (2 * _NSLOT),
)
def _seg_sum(h_hbm, src_hbm, dst_hbm, z_hbm, out_hbm, srcv, dstv, rows, acc,
             *sems):
  gsem = sems[:_NSLOT]
  ssem = sems[_NSLOT:]
  cid = lax.axis_index("c")
  sid = lax.axis_index("s")
  wid = sid * _NC + cid

  def start_gather(ci, s):
    pltpu.async_copy(h_hbm.at[srcv.at[pl.ds(ci * _C, _C)]], rows.at[s],
                     gsem[s])

  def wait_gather(ci, s):
    pltpu.make_async_copy(h_hbm.at[srcv.at[pl.ds(ci * _C, _C)]], rows.at[s],
                          gsem[s]).wait()

  def start_scatter(ci, s):
    pltpu.async_copy(rows.at[s], acc.at[dstv.at[ci]], ssem[s], add=True)

  def wait_scatter(s):
    pltpu.make_async_copy(rows.at[s], acc.at[dstv.at[0]], ssem[s]).wait()

  pltpu.sync_copy(src_hbm.at[pl.ds(wid * _EPW, _EPW)], srcv)
  pltpu.sync_copy(dst_hbm.at[wid], dstv)
  for s in range(_NSLOT):
    start_gather(s, s)
  zoff = jnp.minimum(sid * _ZR, _ZMAX)
  pltpu.sync_copy(z_hbm, acc.at[pl.ds(zoff, _ZR)])
  plsc.subcore_barrier()

  @pl.loop(0, _NCH - 1, step=_NSLOT)
  def _(i):
    for s in range(_NSLOT):
      wait_gather(i + s, s)
      start_scatter(i + s, s)
    for s in range(_NSLOT):
      j = i + _NSLOT + s

      @pl.when(j < _NCH)
      def _():
        wait_scatter(s)
        start_gather(j, s)

  wait_gather(_NCH - 1, 0)
  start_scatter(_NCH - 1, 0)
  for s in range(_NSLOT):
    wait_scatter(s)

  plsc.subcore_barrier()
  pltpu.sync_copy(acc.at[pl.ds(zoff, _ZR)],
                  out_hbm.at[cid, pl.ds(zoff, _ZR)])


def _tc_pre(xp, wrel, wroot, b):
  n, d_in = xp.shape
  grid = (n // _BN,)

  def body(x_ref, wrel_ref, wroot_ref, b_ref, y_ref, r_ref):
    xb = x_ref[...]
    y_ref[...] = jnp.dot(xb, wrel_ref[...],
                         preferred_element_type=jnp.float32, precision=_PREC)
    r_ref[...] = jnp.dot(xb, wroot_ref[...],
                         preferred_element_type=jnp.float32,
                         precision=_PREC) + b_ref[...]

  return pl.pallas_call(
      body,
      grid=grid,
      in_specs=[
          pl.BlockSpec((_BN, d_in), lambda i: (i, 0)),
          pl.BlockSpec((d_in, _H), lambda i: (0, 0)),
          pl.BlockSpec((d_in, _H), lambda i: (0, 0)),
          pl.BlockSpec((1, _H), lambda i: (0, 0)),
      ],
      out_specs=[
          pl.BlockSpec((_BN, _H), lambda i: (i, 0)),
          pl.BlockSpec((_BN, _H), lambda i: (i, 0)),
      ],
      out_shape=[
          jax.ShapeDtypeStruct((n, _H), jnp.float32),
          jax.ShapeDtypeStruct((n, _H), jnp.float32),
      ],
  )(xp, wrel, wroot, b)


def _tc_mid(p, r, wrel, wroot, b):
  n = r.shape[0]
  grid = (n // _BN,)

  def body(p_ref, r_ref, wrel_ref, wroot_ref, b_ref, y_ref, rn_ref):
    h = jnp.maximum(p_ref[0] + p_ref[1] + r_ref[...], 0.0)
    y_ref[...] = jnp.dot(h, wrel_ref[...],
                         preferred_element_type=jnp.float32, precision=_PREC)
    rn_ref[...] = jnp.dot(h, wroot_ref[...],
                          preferred_element_type=jnp.float32,
                          precision=_PREC) + b_ref[...]

  return pl.pallas_call(
      body,
      grid=grid,
      in_specs=[
          pl.BlockSpec((_NC, _BN, _H), lambda i: (0, i, 0)),
          pl.BlockSpec((_BN, _H), lambda i: (i, 0)),
          pl.BlockSpec((_H, _H), lambda i: (0, 0)),
          pl.BlockSpec((_H, _H), lambda i: (0, 0)),
          pl.BlockSpec((1, _H), lambda i: (0, 0)),
      ],
      out_specs=[
          pl.BlockSpec((_BN, _H), lambda i: (i, 0)),
          pl.BlockSpec((_BN, _H), lambda i: (i, 0)),
      ],
      out_shape=[
          jax.ShapeDtypeStruct((n, _H), jnp.float32),
          jax.ShapeDtypeStruct((n, _H), jnp.float32),
      ],
  )(p, r, wrel, wroot, b)


def _tc_final(p, r, batch3d, wimp, bimp, wsta, wstb, bst):
  n = r.shape[0]
  grid_n = n // _BN

  def body(p_ref, r_ref, batch_ref, wimp_ref, bimp_ref, wsta_ref, wstb_ref,
           bst_ref, imp_ref, st_ref, psum, pcnt):
    i = pl.program_id(0)

    @pl.when(i == 0)
    def _():
      psum[...] = jnp.zeros_like(psum)
      pcnt[...] = jnp.zeros_like(pcnt)

    h3 = p_ref[0] + p_ref[1] + r_ref[...]
    bvec = batch_ref[0]
    sel = (lax.broadcasted_iota(jnp.int32, (_G, _BN), 0) == bvec)
    sel = sel.astype(jnp.float32)
    psum[...] += jnp.dot(sel, h3, preferred_element_type=jnp.float32,
                         precision=_PREC)
    pcnt[...] += jnp.sum(sel, axis=1, keepdims=True)

    @pl.when(i == grid_n - 1)
    def _():
      pooled = psum[...] / jnp.maximum(pcnt[...], 1.0)
      imp = jnp.dot(pooled, wimp_ref[...], preferred_element_type=jnp.float32,
                    precision=_PREC) + bimp_ref[...]
      st = jnp.dot(pooled, wsta_ref[...], preferred_element_type=jnp.float32,
                   precision=_PREC)
      st += jnp.dot(imp, wstb_ref[...], preferred_element_type=jnp.float32,
                    precision=_PREC)
      st += bst_ref[...]
      imp_ref[...] = imp
      st_ref[...] = st

  return pl.pallas_call(
      body,
      grid=(grid_n,),
      in_specs=[
          pl.BlockSpec((_NC, _BN, _H), lambda i: (0, i, 0)),
          pl.BlockSpec((_BN, _H), lambda i: (i, 0)),
          pl.BlockSpec((1, 1, _BN), lambda i: (i, 0, 0)),
          pl.BlockSpec((_H, 3), lambda i: (0, 0)),
          pl.BlockSpec((1, 3), lambda i: (0, 0)),
          pl.BlockSpec((_H, 3), lambda i: (0, 0)),
          pl.BlockSpec((3, 3), lambda i: (0, 0)),
          pl.BlockSpec((1, 3), lambda i: (0, 0)),
      ],
      out_specs=[
          pl.BlockSpec((_G, 3), lambda i: (0, 0)),
          pl.BlockSpec((_G, 3), lambda i: (0, 0)),
      ],
      out_shape=[
          jax.ShapeDtypeStruct((_G, 3), jnp.float32),
          jax.ShapeDtypeStruct((_G, 3), jnp.float32),
      ],
      scratch_shapes=[
          pltpu.VMEM((_G, _H), jnp.float32),
          pltpu.VMEM((_G, 1), jnp.float32),
      ],
  )(p, r, batch3d, wimp, bimp, wsta, wstb, bst)


def kernel(x, edge_index, batch, Wrel1, Wroot1, b1, Wrel2, Wroot2, b2,
           Wrel3, Wroot3, b3, Wimp, bimp, Wst, bst):
  src_flat = edge_index[0]
  dst3d = edge_index[1].reshape(_NW, _NCH, _C)
  xp = jnp.pad(x, ((0, 0), (0, 8 - x.shape[1])))
  wrel1p = jnp.pad(Wrel1, ((0, 8 - Wrel1.shape[0]), (0, 0)))
  wroot1p = jnp.pad(Wroot1, ((0, 8 - Wroot1.shape[0]), (0, 0)))
  zeros = jnp.zeros((_ZR, _H), jnp.float32)
  batch3d = batch.reshape(_N // _BN, 1, _BN)

  y1, r1 = _tc_pre(xp, wrel1p, wroot1p, b1.reshape(1, _H))
  p1 = _seg_sum(y1, src_flat, dst3d, zeros)
  y2, r2 = _tc_mid(p1, r1, Wrel2, Wroot2, b2.reshape(1, _H))
  p2 = _seg_sum(y2, src_flat, dst3d, zeros)
  y3, r3 = _tc_mid(p2, r2, Wrel3, Wroot3, b3.reshape(1, _H))
  p3 = _seg_sum(y3, src_flat, dst3d, zeros)
  imp, st = _tc_final(p3, r3, batch3d, Wimp, bimp.reshape(1, 3),
                      Wst[:_H], Wst[_H:], bst.reshape(1, 3))
  return (imp, st)


# speedup vs baseline: 3.4143x; 1.2250x over previous
import functools

import jax
import jax.numpy as jnp
from jax import lax
from jax.experimental import pallas as pl
from jax.experimental.pallas import tpu as pltpu
from jax.experimental.pallas import tpu_sc as plsc

_N = 10000
_E = 320000
_H = 128
_G = 64

_NC = 2
_NS = 16
_NW = _NC * _NS

_C = 80
_EPW = _E // _NW
_NCH = _EPW // _C
_ZR = 632
_ZMAX = _N - _ZR

_BN = 2000
_PREC = jax.lax.Precision.DEFAULT

_mesh = plsc.VectorSubcoreMesh(core_axis_name="c", subcore_axis_name="s")


_NSLOT = 3


@functools.partial(
    pl.kernel,
    mesh=_mesh,
    out_type=jax.ShapeDtypeStruct((_NC, _N, _H), jnp.float32),
    scratch_types=[
        pltpu.VMEM((_EPW,), jnp.int32),
        pltpu.VMEM((_EPW,), jnp.int32),
        pltpu.VMEM((_NSLOT, _C, _H), jnp.float32),
        pltpu.VMEM_SHARED((_N, _H), jnp.float32),
    ] + [pltpu.SemaphoreType.DMA] * (2 * _NSLOT),
)
def _seg_sum(h_hbm, src_hbm, dst_hbm, z_hbm, out_hbm, srcv, dstv, rows, acc,
             *sems):
  gsem = sems[:_NSLOT]
  ssem = sems[_NSLOT:]
  cid = lax.axis_index("c")
  sid = lax.axis_index("s")
  wid = sid * _NC + cid

  def start_gather(ci, s):
    pltpu.async_copy(h_hbm.at[srcv.at[pl.ds(ci * _C, _C)]], rows.at[s],
                     gsem[s])

  def wait_gather(ci, s):
    pltpu.make_async_copy(h_hbm.at[srcv.at[pl.ds(ci * _C, _C)]], rows.at[s],
                          gsem[s]).wait()

  def start_scatter(ci, s):
    pltpu.async_copy(rows.at[s], acc.at[dstv.at[pl.ds(ci * _C, _C)]],
                     ssem[s], add=True)

  def wait_scatter(s):
    pltpu.make_async_copy(rows.at[s], acc.at[dstv.at[pl.ds(0, _C)]],
                          ssem[s]).wait()

  pltpu.sync_copy(src_hbm.at[pl.ds(wid * _EPW, _EPW)], srcv)
  pltpu.sync_copy(dst_hbm.at[pl.ds(wid * _EPW, _EPW)], dstv)
  for s in range(_NSLOT):
    start_gather(s, s)
  zoff = jnp.minimum(sid * _ZR, _ZMAX)
  pltpu.sync_copy(z_hbm, acc.at[pl.ds(zoff, _ZR)])
  plsc.subcore_barrier()

  @pl.loop(0, _NCH, step=_NSLOT)
  def _(i):
    for s in range(_NSLOT):
      ci = i + s

      @pl.when(ci < _NCH)
      def _():
        wait_gather(ci, s)
        start_scatter(ci, s)
    for s in range(_NSLOT):
      j = i + _NSLOT + s

      @pl.when(j < _NCH)
      def _():
        wait_scatter(s)
        start_gather(j, s)

  for s in range(_NSLOT):
    wait_scatter(s)

  plsc.subcore_barrier()
  pltpu.sync_copy(acc.at[pl.ds(zoff, _ZR)],
                  out_hbm.at[cid, pl.ds(zoff, _ZR)])


def _tc_pre(xp, wrel, wroot, b):
  n, d_in = xp.shape
  grid = (n // _BN,)

  def body(x_ref, wrel_ref, wroot_ref, b_ref, y_ref, r_ref):
    xb = x_ref[...]
    y_ref[...] = jnp.dot(xb, wrel_ref[...],
                         preferred_element_type=jnp.float32, precision=_PREC)
    r_ref[...] = jnp.dot(xb, wroot_ref[...],
                         preferred_element_type=jnp.float32,
                         precision=_PREC) + b_ref[...]

  return pl.pallas_call(
      body,
      grid=grid,
      in_specs=[
          pl.BlockSpec((_BN, d_in), lambda i: (i, 0)),
          pl.BlockSpec((d_in, _H), lambda i: (0, 0)),
          pl.BlockSpec((d_in, _H), lambda i: (0, 0)),
          pl.BlockSpec((1, _H), lambda i: (0, 0)),
      ],
      out_specs=[
          pl.BlockSpec((_BN, _H), lambda i: (i, 0)),
          pl.BlockSpec((_BN, _H), lambda i: (i, 0)),
      ],
      out_shape=[
          jax.ShapeDtypeStruct((n, _H), jnp.float32),
          jax.ShapeDtypeStruct((n, _H), jnp.float32),
      ],
  )(xp, wrel, wroot, b)


def _tc_mid(p, r, wrel, wroot, b):
  n = r.shape[0]
  grid = (n // _BN,)

  def body(p_ref, r_ref, wrel_ref, wroot_ref, b_ref, y_ref, rn_ref):
    h = jnp.maximum(p_ref[0] + p_ref[1] + r_ref[...], 0.0)
    y_ref[...] = jnp.dot(h, wrel_ref[...],
                         preferred_element_type=jnp.float32, precision=_PREC)
    rn_ref[...] = jnp.dot(h, wroot_ref[...],
                          preferred_element_type=jnp.float32,
                          precision=_PREC) + b_ref[...]

  return pl.pallas_call(
      body,
      grid=grid,
      in_specs=[
          pl.BlockSpec((_NC, _BN, _H), lambda i: (0, i, 0)),
          pl.BlockSpec((_BN, _H), lambda i: (i, 0)),
          pl.BlockSpec((_H, _H), lambda i: (0, 0)),
          pl.BlockSpec((_H, _H), lambda i: (0, 0)),
          pl.BlockSpec((1, _H), lambda i: (0, 0)),
      ],
      out_specs=[
          pl.BlockSpec((_BN, _H), lambda i: (i, 0)),
          pl.BlockSpec((_BN, _H), lambda i: (i, 0)),
      ],
      out_shape=[
          jax.ShapeDtypeStruct((n, _H), jnp.float32),
          jax.ShapeDtypeStruct((n, _H), jnp.float32),
      ],
  )(p, r, wrel, wroot, b)


def _tc_final(p, r, batch3d, wimp, bimp, wsta, wstb, bst):
  n = r.shape[0]
  grid_n = n // _BN

  def body(p_ref, r_ref, batch_ref, wimp_ref, bimp_ref, wsta_ref, wstb_ref,
           bst_ref, imp_ref, st_ref, psum, pcnt):
    i = pl.program_id(0)

    @pl.when(i == 0)
    def _():
      psum[...] = jnp.zeros_like(psum)
      pcnt[...] = jnp.zeros_like(pcnt)

    h3 = p_ref[0] + p_ref[1] + r_ref[...]
    bvec = batch_ref[0]
    sel = (lax.broadcasted_iota(jnp.int32, (_G, _BN), 0) == bvec)
    sel = sel.astype(jnp.float32)
    psum[...] += jnp.dot(sel, h3, preferred_element_type=jnp.float32,
                         precision=_PREC)
    pcnt[...] += jnp.sum(sel, axis=1, keepdims=True)

    @pl.when(i == grid_n - 1)
    def _():
      pooled = psum[...] / jnp.maximum(pcnt[...], 1.0)
      imp = jnp.dot(pooled, wimp_ref[...], preferred_element_type=jnp.float32,
                    precision=_PREC) + bimp_ref[...]
      st = jnp.dot(pooled, wsta_ref[...], preferred_element_type=jnp.float32,
                   precision=_PREC)
      st += jnp.dot(imp, wstb_ref[...], preferred_element_type=jnp.float32,
                    precision=_PREC)
      st += bst_ref[...]
      imp_ref[...] = imp
      st_ref[...] = st

  return pl.pallas_call(
      body,
      grid=(grid_n,),
      in_specs=[
          pl.BlockSpec((_NC, _BN, _H), lambda i: (0, i, 0)),
          pl.BlockSpec((_BN, _H), lambda i: (i, 0)),
          pl.BlockSpec((1, 1, _BN), lambda i: (i, 0, 0)),
          pl.BlockSpec((_H, 3), lambda i: (0, 0)),
          pl.BlockSpec((1, 3), lambda i: (0, 0)),
          pl.BlockSpec((_H, 3), lambda i: (0, 0)),
          pl.BlockSpec((3, 3), lambda i: (0, 0)),
          pl.BlockSpec((1, 3), lambda i: (0, 0)),
      ],
      out_specs=[
          pl.BlockSpec((_G, 3), lambda i: (0, 0)),
          pl.BlockSpec((_G, 3), lambda i: (0, 0)),
      ],
      out_shape=[
          jax.ShapeDtypeStruct((_G, 3), jnp.float32),
          jax.ShapeDtypeStruct((_G, 3), jnp.float32),
      ],
      scratch_shapes=[
          pltpu.VMEM((_G, _H), jnp.float32),
          pltpu.VMEM((_G, 1), jnp.float32),
      ],
  )(p, r, batch3d, wimp, bimp, wsta, wstb, bst)


def kernel(x, edge_index, batch, Wrel1, Wroot1, b1, Wrel2, Wroot2, b2,
           Wrel3, Wroot3, b3, Wimp, bimp, Wst, bst):
  src_flat = edge_index[0]
  dst_flat = edge_index[1]
  xp = jnp.pad(x, ((0, 0), (0, 8 - x.shape[1])))
  wrel1p = jnp.pad(Wrel1, ((0, 8 - Wrel1.shape[0]), (0, 0)))
  wroot1p = jnp.pad(Wroot1, ((0, 8 - Wroot1.shape[0]), (0, 0)))
  zeros = jnp.zeros((_ZR, _H), jnp.float32)
  batch3d = batch.reshape(_N // _BN, 1, _BN)

  y1, r1 = _tc_pre(xp, wrel1p, wroot1p, b1.reshape(1, _H))
  p1 = _seg_sum(y1, src_flat, dst_flat, zeros)
  y2, r2 = _tc_mid(p1, r1, Wrel2, Wroot2, b2.reshape(1, _H))
  p2 = _seg_sum(y2, src_flat, dst_flat, zeros)
  y3, r3 = _tc_mid(p2, r2, Wrel3, Wroot3, b3.reshape(1, _H))
  p3 = _seg_sum(y3, src_flat, dst_flat, zeros)
  imp, st = _tc_final(p3, r3, batch3d, Wimp, bimp.reshape(1, 3),
                      Wst[:_H], Wst[_H:], bst.reshape(1, 3))
  return (imp, st)


# speedup vs baseline: 3.6010x; 1.0547x over previous
import functools

import jax
import jax.numpy as jnp
from jax import lax
from jax.experimental import pallas as pl
from jax.experimental.pallas import tpu as pltpu
from jax.experimental.pallas import tpu_sc as plsc

_N = 10000
_E = 320000
_H = 128
_G = 64

_NC = 2
_NS = 16
_NW = _NC * _NS

_C = 40
_EPW = _E // _NW
_NCH = _EPW // _C
_ZR = 632
_ZMAX = _N - _ZR

_BN = 2000
_PREC = jax.lax.Precision.DEFAULT

_mesh = plsc.VectorSubcoreMesh(core_axis_name="c", subcore_axis_name="s")


_NSLOT = 5


@functools.partial(
    pl.kernel,
    mesh=_mesh,
    out_type=jax.ShapeDtypeStruct((_NC, _N, _H), jnp.float32),
    scratch_types=[
        pltpu.VMEM((_EPW,), jnp.int32),
        pltpu.VMEM((_EPW,), jnp.int32),
        pltpu.VMEM((_NSLOT, _C, _H), jnp.float32),
        pltpu.VMEM_SHARED((_N, _H), jnp.float32),
    ] + [pltpu.SemaphoreType.DMA] * (2 * _NSLOT),
)
def _seg_sum(h_hbm, src_hbm, dst_hbm, z_hbm, out_hbm, srcv, dstv, rows, acc,
             *sems):
  gsem = sems[:_NSLOT]
  ssem = sems[_NSLOT:]
  cid = lax.axis_index("c")
  sid = lax.axis_index("s")
  wid = sid * _NC + cid

  def start_gather(ci, s):
    pltpu.async_copy(h_hbm.at[srcv.at[pl.ds(ci * _C, _C)]], rows.at[s],
                     gsem[s])

  def wait_gather(ci, s):
    pltpu.make_async_copy(h_hbm.at[srcv.at[pl.ds(ci * _C, _C)]], rows.at[s],
                          gsem[s]).wait()

  def start_scatter(ci, s):
    pltpu.async_copy(rows.at[s], acc.at[dstv.at[pl.ds(ci * _C, _C)]],
                     ssem[s], add=True)

  def wait_scatter(s):
    pltpu.make_async_copy(rows.at[s], acc.at[dstv.at[pl.ds(0, _C)]],
                          ssem[s]).wait()

  pltpu.sync_copy(src_hbm.at[pl.ds(wid * _EPW, _EPW)], srcv)
  pltpu.sync_copy(dst_hbm.at[pl.ds(wid * _EPW, _EPW)], dstv)
  for s in range(_NSLOT):
    start_gather(s, s)
  zoff = jnp.minimum(sid * _ZR, _ZMAX)
  pltpu.sync_copy(z_hbm, acc.at[pl.ds(zoff, _ZR)])
  plsc.subcore_barrier()

  @pl.loop(0, _NCH, step=_NSLOT)
  def _(i):
    for s in range(_NSLOT):
      ci = i + s

      @pl.when(ci < _NCH)
      def _():
        wait_gather(ci, s)
        start_scatter(ci, s)
    for s in range(_NSLOT):
      j = i + _NSLOT + s

      @pl.when(j < _NCH)
      def _():
        wait_scatter(s)
        start_gather(j, s)

  for s in range(_NSLOT):
    wait_scatter(s)

  plsc.subcore_barrier()
  pltpu.sync_copy(acc.at[pl.ds(zoff, _ZR)],
                  out_hbm.at[cid, pl.ds(zoff, _ZR)])


def _tc_pre(xp, wrel, wroot, b):
  n, d_in = xp.shape
  grid = (n // _BN,)

  def body(x_ref, wrel_ref, wroot_ref, b_ref, y_ref, r_ref):
    xb = x_ref[...]
    y_ref[...] = jnp.dot(xb, wrel_ref[...],
                         preferred_element_type=jnp.float32, precision=_PREC)
    r_ref[...] = jnp.dot(xb, wroot_ref[...],
                         preferred_element_type=jnp.float32,
                         precision=_PREC) + b_ref[...]

  return pl.pallas_call(
      body,
      grid=grid,
      in_specs=[
          pl.BlockSpec((_BN, d_in), lambda i: (i, 0)),
          pl.BlockSpec((d_in, _H), lambda i: (0, 0)),
          pl.BlockSpec((d_in, _H), lambda i: (0, 0)),
          pl.BlockSpec((1, _H), lambda i: (0, 0)),
      ],
      out_specs=[
          pl.BlockSpec((_BN, _H), lambda i: (i, 0)),
          pl.BlockSpec((_BN, _H), lambda i: (i, 0)),
      ],
      out_shape=[
          jax.ShapeDtypeStruct((n, _H), jnp.float32),
          jax.ShapeDtypeStruct((n, _H), jnp.float32),
      ],
  )(xp, wrel, wroot, b)


def _tc_mid(p, r, wrel, wroot, b):
  n = r.shape[0]
  grid = (n // _BN,)

  def body(p_ref, r_ref, wrel_ref, wroot_ref, b_ref, y_ref, rn_ref):
    h = jnp.maximum(p_ref[0] + p_ref[1] + r_ref[...], 0.0)
    y_ref[...] = jnp.dot(h, wrel_ref[...],
                         preferred_element_type=jnp.float32, precision=_PREC)
    rn_ref[...] = jnp.dot(h, wroot_ref[...],
                          preferred_element_type=jnp.float32,
                          precision=_PREC) + b_ref[...]

  return pl.pallas_call(
      body,
      grid=grid,
      in_specs=[
          pl.BlockSpec((_NC, _BN, _H), lambda i: (0, i, 0)),
          pl.BlockSpec((_BN, _H), lambda i: (i, 0)),
          pl.BlockSpec((_H, _H), lambda i: (0, 0)),
          pl.BlockSpec((_H, _H), lambda i: (0, 0)),
          pl.BlockSpec((1, _H), lambda i: (0, 0)),
      ],
      out_specs=[
          pl.BlockSpec((_BN, _H), lambda i: (i, 0)),
          pl.BlockSpec((_BN, _H), lambda i: (i, 0)),
      ],
      out_shape=[
          jax.ShapeDtypeStruct((n, _H), jnp.float32),
          jax.ShapeDtypeStruct((n, _H), jnp.float32),
      ],
  )(p, r, wrel, wroot, b)


def _tc_final(p, r, batch3d, wimp, bimp, wsta, wstb, bst):
  n = r.shape[0]
  grid_n = n // _BN

  def body(p_ref, r_ref, batch_ref, wimp_ref, bimp_ref, wsta_ref, wstb_ref,
           bst_ref, imp_ref, st_ref, psum, pcnt):
    i = pl.program_id(0)

    @pl.when(i == 0)
    def _():
      psum[...] = jnp.zeros_like(psum)
      pcnt[...] = jnp.zeros_like(pcnt)

    h3 = p_ref[0] + p_ref[1] + r_ref[...]
    bvec = batch_ref[0]
    sel = (lax.broadcasted_iota(jnp.int32, (_G, _BN), 0) == bvec)
    sel = sel.astype(jnp.float32)
    psum[...] += jnp.dot(sel, h3, preferred_element_type=jnp.float32,
                         precision=_PREC)
    pcnt[...] += jnp.sum(sel, axis=1, keepdims=True)

    @pl.when(i == grid_n - 1)
    def _():
      pooled = psum[...] / jnp.maximum(pcnt[...], 1.0)
      imp = jnp.dot(pooled, wimp_ref[...], preferred_element_type=jnp.float32,
                    precision=_PREC) + bimp_ref[...]
      st = jnp.dot(pooled, wsta_ref[...], preferred_element_type=jnp.float32,
                   precision=_PREC)
      st += jnp.dot(imp, wstb_ref[...], preferred_element_type=jnp.float32,
                    precision=_PREC)
      st += bst_ref[...]
      imp_ref[...] = imp
      st_ref[...] = st

  return pl.pallas_call(
      body,
      grid=(grid_n,),
      in_specs=[
          pl.BlockSpec((_NC, _BN, _H), lambda i: (0, i, 0)),
          pl.BlockSpec((_BN, _H), lambda i: (i, 0)),
          pl.BlockSpec((1, 1, _BN), lambda i: (i, 0, 0)),
          pl.BlockSpec((_H, 3), lambda i: (0, 0)),
          pl.BlockSpec((1, 3), lambda i: (0, 0)),
          pl.BlockSpec((_H, 3), lambda i: (0, 0)),
          pl.BlockSpec((3, 3), lambda i: (0, 0)),
          pl.BlockSpec((1, 3), lambda i: (0, 0)),
      ],
      out_specs=[
          pl.BlockSpec((_G, 3), lambda i: (0, 0)),
          pl.BlockSpec((_G, 3), lambda i: (0, 0)),
      ],
      out_shape=[
          jax.ShapeDtypeStruct((_G, 3), jnp.float32),
          jax.ShapeDtypeStruct((_G, 3), jnp.float32),
      ],
      scratch_shapes=[
          pltpu.VMEM((_G, _H), jnp.float32),
          pltpu.VMEM((_G, 1), jnp.float32),
      ],
  )(p, r, batch3d, wimp, bimp, wsta, wstb, bst)


def kernel(x, edge_index, batch, Wrel1, Wroot1, b1, Wrel2, Wroot2, b2,
           Wrel3, Wroot3, b3, Wimp, bimp, Wst, bst):
  src_flat = edge_index[0]
  dst_flat = edge_index[1]
  xp = jnp.pad(x, ((0, 0), (0, 8 - x.shape[1])))
  wrel1p = jnp.pad(Wrel1, ((0, 8 - Wrel1.shape[0]), (0, 0)))
  wroot1p = jnp.pad(Wroot1, ((0, 8 - Wroot1.shape[0]), (0, 0)))
  zeros = jnp.zeros((_ZR, _H), jnp.float32)
  batch3d = batch.reshape(_N // _BN, 1, _BN)

  y1, r1 = _tc_pre(xp, wrel1p, wroot1p, b1.reshape(1, _H))
  p1 = _seg_sum(y1, src_flat, dst_flat, zeros)
  y2, r2 = _tc_mid(p1, r1, Wrel2, Wroot2, b2.reshape(1, _H))
  p2 = _seg_sum(y2, src_flat, dst_flat, zeros)
  y3, r3 = _tc_mid(p2, r2, Wrel3, Wroot3, b3.reshape(1, _H))
  p3 = _seg_sum(y3, src_flat, dst_flat, zeros)
  imp, st = _tc_final(p3, r3, batch3d, Wimp, bimp.reshape(1, 3),
                      Wst[:_H], Wst[_H:], bst.reshape(1, 3))
  return (imp, st)


# speedup vs baseline: 3.6567x; 1.0155x over previous
import functools

import jax
import jax.numpy as jnp
from jax import lax
from jax.experimental import pallas as pl
from jax.experimental.pallas import tpu as pltpu
from jax.experimental.pallas import tpu_sc as plsc

_N = 10000
_E = 320000
_H = 128
_G = 64

_NC = 2
_NS = 16
_NW = _NC * _NS

_C = 40
_EPW = _E // _NW
_NCH = _EPW // _C
_ZR = 632
_ZMAX = _N - _ZR

_BN = 2000
_PREC = jax.lax.Precision.DEFAULT

_mesh = plsc.VectorSubcoreMesh(core_axis_name="c", subcore_axis_name="s")


_NSLOT = 6


@functools.partial(
    pl.kernel,
    mesh=_mesh,
    out_type=jax.ShapeDtypeStruct((_NC, _N, _H), jnp.float32),
    scratch_types=[
        pltpu.VMEM((_EPW,), jnp.int32),
        pltpu.VMEM((_EPW,), jnp.int32),
        pltpu.VMEM((_NSLOT, _C, _H), jnp.float32),
        pltpu.VMEM_SHARED((_N, _H), jnp.float32),
    ] + [pltpu.SemaphoreType.DMA] * (2 * _NSLOT),
)
def _seg_sum(h_hbm, src_hbm, dst_hbm, z_hbm, out_hbm, srcv, dstv, rows, acc,
             *sems):
  gsem = sems[:_NSLOT]
  ssem = sems[_NSLOT:]
  cid = lax.axis_index("c")
  sid = lax.axis_index("s")
  wid = sid * _NC + cid

  def start_gather(ci, s):
    pltpu.async_copy(h_hbm.at[srcv.at[pl.ds(ci * _C, _C)]], rows.at[s],
                     gsem[s])

  def wait_gather(ci, s):
    pltpu.make_async_copy(h_hbm.at[srcv.at[pl.ds(ci * _C, _C)]], rows.at[s],
                          gsem[s]).wait()

  def start_scatter(ci, s):
    pltpu.async_copy(rows.at[s], acc.at[dstv.at[pl.ds(ci * _C, _C)]],
                     ssem[s], add=True)

  def wait_scatter(s):
    pltpu.make_async_copy(rows.at[s], acc.at[dstv.at[pl.ds(0, _C)]],
                          ssem[s]).wait()

  pltpu.sync_copy(src_hbm.at[pl.ds(wid * _EPW, _EPW)], srcv)
  pltpu.sync_copy(dst_hbm.at[pl.ds(wid * _EPW, _EPW)], dstv)
  for s in range(_NSLOT):
    start_gather(s, s)
  zoff = jnp.minimum(sid * _ZR, _ZMAX)
  pltpu.sync_copy(z_hbm, acc.at[pl.ds(zoff, _ZR)])
  plsc.subcore_barrier()

  @pl.loop(0, _NCH, step=_NSLOT)
  def _(i):
    for s in range(_NSLOT):
      ci = i + s

      @pl.when(ci < _NCH)
      def _():
        wait_gather(ci, s)
        start_scatter(ci, s)
    for s in range(_NSLOT):
      j = i + _NSLOT + s

      @pl.when(j < _NCH)
      def _():
        wait_scatter(s)
        start_gather(j, s)

  for s in range(_NSLOT):
    wait_scatter(s)

  plsc.subcore_barrier()
  pltpu.sync_copy(acc.at[pl.ds(zoff, _ZR)],
                  out_hbm.at[cid, pl.ds(zoff, _ZR)])


def _tc_pre(xp, wrel, wroot, b):
  n, d_in = xp.shape
  grid = (n // _BN,)

  def body(x_ref, wrel_ref, wroot_ref, b_ref, y_ref, r_ref):
    xb = x_ref[...]
    y_ref[...] = jnp.dot(xb, wrel_ref[...],
                         preferred_element_type=jnp.float32, precision=_PREC)
    r_ref[...] = jnp.dot(xb, wroot_ref[...],
                         preferred_element_type=jnp.float32,
                         precision=_PREC) + b_ref[...]

  return pl.pallas_call(
      body,
      grid=grid,
      in_specs=[
          pl.BlockSpec((_BN, d_in), lambda i: (i, 0)),
          pl.BlockSpec((d_in, _H), lambda i: (0, 0)),
          pl.BlockSpec((d_in, _H), lambda i: (0, 0)),
          pl.BlockSpec((1, _H), lambda i: (0, 0)),
      ],
      out_specs=[
          pl.BlockSpec((_BN, _H), lambda i: (i, 0)),
          pl.BlockSpec((_BN, _H), lambda i: (i, 0)),
      ],
      out_shape=[
          jax.ShapeDtypeStruct((n, _H), jnp.float32),
          jax.ShapeDtypeStruct((n, _H), jnp.float32),
      ],
  )(xp, wrel, wroot, b)


def _tc_mid(p, r, wrel, wroot, b):
  n = r.shape[0]
  grid = (n // _BN,)

  def body(p_ref, r_ref, wrel_ref, wroot_ref, b_ref, y_ref, rn_ref):
    h = jnp.maximum(p_ref[0] + p_ref[1] + r_ref[...], 0.0)
    y_ref[...] = jnp.dot(h, wrel_ref[...],
                         preferred_element_type=jnp.float32, precision=_PREC)
    rn_ref[...] = jnp.dot(h, wroot_ref[...],
                          preferred_element_type=jnp.float32,
                          precision=_PREC) + b_ref[...]

  return pl.pallas_call(
      body,
      grid=grid,
      in_specs=[
          pl.BlockSpec((_NC, _BN, _H), lambda i: (0, i, 0)),
          pl.BlockSpec((_BN, _H), lambda i: (i, 0)),
          pl.BlockSpec((_H, _H), lambda i: (0, 0)),
          pl.BlockSpec((_H, _H), lambda i: (0, 0)),
          pl.BlockSpec((1, _H), lambda i: (0, 0)),
      ],
      out_specs=[
          pl.BlockSpec((_BN, _H), lambda i: (i, 0)),
          pl.BlockSpec((_BN, _H), lambda i: (i, 0)),
      ],
      out_shape=[
          jax.ShapeDtypeStruct((n, _H), jnp.float32),
          jax.ShapeDtypeStruct((n, _H), jnp.float32),
      ],
  )(p, r, wrel, wroot, b)


def _tc_final(p, r, batch3d, wimp, bimp, wsta, wstb, bst):
  n = r.shape[0]
  grid_n = n // _BN

  def body(p_ref, r_ref, batch_ref, wimp_ref, bimp_ref, wsta_ref, wstb_ref,
           bst_ref, imp_ref, st_ref, psum, pcnt):
    i = pl.program_id(0)

    @pl.when(i == 0)
    def _():
      psum[...] = jnp.zeros_like(psum)
      pcnt[...] = jnp.zeros_like(pcnt)

    h3 = p_ref[0] + p_ref[1] + r_ref[...]
    bvec = batch_ref[0]
    sel = (lax.broadcasted_iota(jnp.int32, (_G, _BN), 0) == bvec)
    sel = sel.astype(jnp.float32)
    psum[...] += jnp.dot(sel, h3, preferred_element_type=jnp.float32,
                         precision=_PREC)
    pcnt[...] += jnp.sum(sel, axis=1, keepdims=True)

    @pl.when(i == grid_n - 1)
    def _():
      pooled = psum[...] / jnp.maximum(pcnt[...], 1.0)
      imp = jnp.dot(pooled, wimp_ref[...], preferred_element_type=jnp.float32,
                    precision=_PREC) + bimp_ref[...]
      st = jnp.dot(pooled, wsta_ref[...], preferred_element_type=jnp.float32,
                   precision=_PREC)
      st += jnp.dot(imp, wstb_ref[...], preferred_element_type=jnp.float32,
                    precision=_PREC)
      st += bst_ref[...]
      imp_ref[...] = imp
      st_ref[...] = st

  return pl.pallas_call(
      body,
      grid=(grid_n,),
      in_specs=[
          pl.BlockSpec((_NC, _BN, _H), lambda i: (0, i, 0)),
          pl.BlockSpec((_BN, _H), lambda i: (i, 0)),
          pl.BlockSpec((1, 1, _BN), lambda i: (i, 0, 0)),
          pl.BlockSpec((_H, 3), lambda i: (0, 0)),
          pl.BlockSpec((1, 3), lambda i: (0, 0)),
          pl.BlockSpec((_H, 3), lambda i: (0, 0)),
          pl.BlockSpec((3, 3), lambda i: (0, 0)),
          pl.BlockSpec((1, 3), lambda i: (0, 0)),
      ],
      out_specs=[
          pl.BlockSpec((_G, 3), lambda i: (0, 0)),
          pl.BlockSpec((_G, 3), lambda i: (0, 0)),
      ],
      out_shape=[
          jax.ShapeDtypeStruct((_G, 3), jnp.float32),
          jax.ShapeDtypeStruct((_G, 3), jnp.float32),
      ],
      scratch_shapes=[
          pltpu.VMEM((_G, _H), jnp.float32),
          pltpu.VMEM((_G, 1), jnp.float32),
      ],
  )(p, r, batch3d, wimp, bimp, wsta, wstb, bst)


def kernel(x, edge_index, batch, Wrel1, Wroot1, b1, Wrel2, Wroot2, b2,
           Wrel3, Wroot3, b3, Wimp, bimp, Wst, bst):
  src_flat = edge_index[0]
  dst_flat = edge_index[1]
  xp = jnp.pad(x, ((0, 0), (0, 8 - x.shape[1])))
  wrel1p = jnp.pad(Wrel1, ((0, 8 - Wrel1.shape[0]), (0, 0)))
  wroot1p = jnp.pad(Wroot1, ((0, 8 - Wroot1.shape[0]), (0, 0)))
  zeros = jnp.zeros((_ZR, _H), jnp.float32)
  batch3d = batch.reshape(_N // _BN, 1, _BN)

  y1, r1 = _tc_pre(xp, wrel1p, wroot1p, b1.reshape(1, _H))
  p1 = _seg_sum(y1, src_flat, dst_flat, zeros)
  y2, r2 = _tc_mid(p1, r1, Wrel2, Wroot2, b2.reshape(1, _H))
  p2 = _seg_sum(y2, src_flat, dst_flat, zeros)
  y3, r3 = _tc_mid(p2, r2, Wrel3, Wroot3, b3.reshape(1, _H))
  p3 = _seg_sum(y3, src_flat, dst_flat, zeros)
  imp, st = _tc_final(p3, r3, batch3d, Wimp, bimp.reshape(1, 3),
                      Wst[:_H], Wst[_H:], bst.reshape(1, 3))
  return (imp, st)


# speedup vs baseline: 3.7169x; 1.0165x over previous
import functools

import jax
import jax.numpy as jnp
from jax import lax
from jax.experimental import pallas as pl
from jax.experimental.pallas import tpu as pltpu
from jax.experimental.pallas import tpu_sc as plsc

_N = 10000
_E = 320000
_H = 128
_G = 64

_NC = 2
_NS = 16
_NW = _NC * _NS

_C = 40
_EPW = _E // _NW
_NCH = _EPW // _C
_ZR = 640
_ZMAX = _N - _ZR

_BN = 2000
_PREC = jax.lax.Precision.DEFAULT

_mesh = plsc.VectorSubcoreMesh(core_axis_name="c", subcore_axis_name="s")


_NSLOT = 5


@functools.partial(
    pl.kernel,
    mesh=_mesh,
    out_type=jax.ShapeDtypeStruct((_NC, _N, _H), jnp.float32),
    scratch_types=[
        pltpu.VMEM((_EPW,), jnp.int32),
        pltpu.VMEM((_EPW,), jnp.int32),
        pltpu.VMEM((_NSLOT, _C, _H), jnp.float32),
        pltpu.VMEM((16, _H), jnp.float32),
        pltpu.VMEM_SHARED((_N, _H), jnp.float32),
    ] + [pltpu.SemaphoreType.DMA] * (2 * _NSLOT + 1),
)
def _seg_sum(h_hbm, src_hbm, dst_hbm, z_hbm, out_hbm, srcv, dstv, rows, zbuf,
             acc, *sems):
  gsem = sems[:_NSLOT]
  ssem = sems[_NSLOT:2 * _NSLOT]
  zsem = sems[2 * _NSLOT]
  cid = lax.axis_index("c")
  sid = lax.axis_index("s")
  wid = sid * _NC + cid

  def start_gather(ci, s):
    pltpu.async_copy(h_hbm.at[srcv.at[pl.ds(ci * _C, _C)]], rows.at[s],
                     gsem[s])

  def wait_gather(ci, s):
    pltpu.make_async_copy(h_hbm.at[srcv.at[pl.ds(ci * _C, _C)]], rows.at[s],
                          gsem[s]).wait()

  def start_scatter(ci, s):
    pltpu.async_copy(rows.at[s], acc.at[dstv.at[pl.ds(ci * _C, _C)]],
                     ssem[s], add=True)

  def wait_scatter(s):
    pltpu.make_async_copy(rows.at[s], acc.at[dstv.at[pl.ds(0, _C)]],
                          ssem[s]).wait()

  zoff = jnp.minimum(sid * _ZR, _ZMAX)
  pltpu.sync_copy(z_hbm, zbuf)
  _NZB = _ZR // 16

  @pl.loop(0, _NZB)
  def _(k):
    pltpu.async_copy(zbuf, acc.at[pl.ds(zoff + k * 16, 16)], zsem)

  pltpu.sync_copy(src_hbm.at[pl.ds(wid * _EPW, _EPW)], srcv)
  pltpu.sync_copy(dst_hbm.at[pl.ds(wid * _EPW, _EPW)], dstv)
  for s in range(_NSLOT):
    start_gather(s, s)

  @pl.loop(0, _NZB)
  def _(k):
    pltpu.make_async_copy(zbuf, acc.at[pl.ds(zoff, 16)], zsem).wait()

  plsc.subcore_barrier()

  @pl.loop(0, _NCH, step=_NSLOT)
  def _(i):
    for s in range(_NSLOT):
      ci = i + s

      @pl.when(ci < _NCH)
      def _():
        wait_gather(ci, s)
        start_scatter(ci, s)
    for s in range(_NSLOT):
      j = i + _NSLOT + s

      @pl.when(j < _NCH)
      def _():
        wait_scatter(s)
        start_gather(j, s)

  for s in range(_NSLOT):
    wait_scatter(s)

  plsc.subcore_barrier()
  pltpu.sync_copy(acc.at[pl.ds(zoff, _ZR)],
                  out_hbm.at[cid, pl.ds(zoff, _ZR)])


def _tc_pre(xp, wrel, wroot, b):
  n, d_in = xp.shape
  grid = (n // _BN,)

  def body(x_ref, wrel_ref, wroot_ref, b_ref, y_ref, r_ref):
    xb = x_ref[...]
    y_ref[...] = jnp.dot(xb, wrel_ref[...],
                         preferred_element_type=jnp.float32, precision=_PREC)
    r_ref[...] = jnp.dot(xb, wroot_ref[...],
                         preferred_element_type=jnp.float32,
                         precision=_PREC) + b_ref[...]

  return pl.pallas_call(
      body,
      grid=grid,
      in_specs=[
          pl.BlockSpec((_BN, d_in), lambda i: (i, 0)),
          pl.BlockSpec((d_in, _H), lambda i: (0, 0)),
          pl.BlockSpec((d_in, _H), lambda i: (0, 0)),
          pl.BlockSpec((1, _H), lambda i: (0, 0)),
      ],
      out_specs=[
          pl.BlockSpec((_BN, _H), lambda i: (i, 0)),
          pl.BlockSpec((_BN, _H), lambda i: (i, 0)),
      ],
      out_shape=[
          jax.ShapeDtypeStruct((n, _H), jnp.float32),
          jax.ShapeDtypeStruct((n, _H), jnp.float32),
      ],
  )(xp, wrel, wroot, b)


def _tc_mid(p, r, wrel, wroot, b):
  n = r.shape[0]
  grid = (n // _BN,)

  def body(p_ref, r_ref, wrel_ref, wroot_ref, b_ref, y_ref, rn_ref):
    h = jnp.maximum(p_ref[0] + p_ref[1] + r_ref[...], 0.0)
    y_ref[...] = jnp.dot(h, wrel_ref[...],
                         preferred_element_type=jnp.float32, precision=_PREC)
    rn_ref[...] = jnp.dot(h, wroot_ref[...],
                          preferred_element_type=jnp.float32,
                          precision=_PREC) + b_ref[...]

  return pl.pallas_call(
      body,
      grid=grid,
      in_specs=[
          pl.BlockSpec((_NC, _BN, _H), lambda i: (0, i, 0)),
          pl.BlockSpec((_BN, _H), lambda i: (i, 0)),
          pl.BlockSpec((_H, _H), lambda i: (0, 0)),
          pl.BlockSpec((_H, _H), lambda i: (0, 0)),
          pl.BlockSpec((1, _H), lambda i: (0, 0)),
      ],
      out_specs=[
          pl.BlockSpec((_BN, _H), lambda i: (i, 0)),
          pl.BlockSpec((_BN, _H), lambda i: (i, 0)),
      ],
      out_shape=[
          jax.ShapeDtypeStruct((n, _H), jnp.float32),
          jax.ShapeDtypeStruct((n, _H), jnp.float32),
      ],
  )(p, r, wrel, wroot, b)


def _tc_final(p, r, batch3d, wimp, bimp, wsta, wstb, bst):
  n = r.shape[0]
  grid_n = n // _BN

  def body(p_ref, r_ref, batch_ref, wimp_ref, bimp_ref, wsta_ref, wstb_ref,
           bst_ref, imp_ref, st_ref, psum, pcnt):
    i = pl.program_id(0)

    @pl.when(i == 0)
    def _():
      psum[...] = jnp.zeros_like(psum)
      pcnt[...] = jnp.zeros_like(pcnt)

    h3 = p_ref[0] + p_ref[1] + r_ref[...]
    bvec = batch_ref[0]
    sel = (lax.broadcasted_iota(jnp.int32, (_G, _BN), 0) == bvec)
    sel = sel.astype(jnp.float32)
    psum[...] += jnp.dot(sel, h3, preferred_element_type=jnp.float32,
                         precision=_PREC)
    pcnt[...] += jnp.sum(sel, axis=1, keepdims=True)

    @pl.when(i == grid_n - 1)
    def _():
      pooled = psum[...] / jnp.maximum(pcnt[...], 1.0)
      imp = jnp.dot(pooled, wimp_ref[...], preferred_element_type=jnp.float32,
                    precision=_PREC) + bimp_ref[...]
      st = jnp.dot(pooled, wsta_ref[...], preferred_element_type=jnp.float32,
                   precision=_PREC)
      st += jnp.dot(imp, wstb_ref[...], preferred_element_type=jnp.float32,
                    precision=_PREC)
      st += bst_ref[...]
      imp_ref[...] = imp
      st_ref[...] = st

  return pl.pallas_call(
      body,
      grid=(grid_n,),
      in_specs=[
          pl.BlockSpec((_NC, _BN, _H), lambda i: (0, i, 0)),
          pl.BlockSpec((_BN, _H), lambda i: (i, 0)),
          pl.BlockSpec((1, 1, _BN), lambda i: (i, 0, 0)),
          pl.BlockSpec((_H, 3), lambda i: (0, 0)),
          pl.BlockSpec((1, 3), lambda i: (0, 0)),
          pl.BlockSpec((_H, 3), lambda i: (0, 0)),
          pl.BlockSpec((3, 3), lambda i: (0, 0)),
          pl.BlockSpec((1, 3), lambda i: (0, 0)),
      ],
      out_specs=[
          pl.BlockSpec((_G, 3), lambda i: (0, 0)),
          pl.BlockSpec((_G, 3), lambda i: (0, 0)),
      ],
      out_shape=[
          jax.ShapeDtypeStruct((_G, 3), jnp.float32),
          jax.ShapeDtypeStruct((_G, 3), jnp.float32),
      ],
      scratch_shapes=[
          pltpu.VMEM((_G, _H), jnp.float32),
          pltpu.VMEM((_G, 1), jnp.float32),
      ],
  )(p, r, batch3d, wimp, bimp, wsta, wstb, bst)


def kernel(x, edge_index, batch, Wrel1, Wroot1, b1, Wrel2, Wroot2, b2,
           Wrel3, Wroot3, b3, Wimp, bimp, Wst, bst):
  src_flat = edge_index[0]
  dst_flat = edge_index[1]
  xp = jnp.pad(x, ((0, 0), (0, 8 - x.shape[1])))
  wrel1p = jnp.pad(Wrel1, ((0, 8 - Wrel1.shape[0]), (0, 0)))
  wroot1p = jnp.pad(Wroot1, ((0, 8 - Wroot1.shape[0]), (0, 0)))
  zeros = jnp.zeros((16, _H), jnp.float32)
  batch3d = batch.reshape(_N // _BN, 1, _BN)

  y1, r1 = _tc_pre(xp, wrel1p, wroot1p, b1.reshape(1, _H))
  p1 = _seg_sum(y1, src_flat, dst_flat, zeros)
  y2, r2 = _tc_mid(p1, r1, Wrel2, Wroot2, b2.reshape(1, _H))
  p2 = _seg_sum(y2, src_flat, dst_flat, zeros)
  y3, r3 = _tc_mid(p2, r2, Wrel3, Wroot3, b3.reshape(1, _H))
  p3 = _seg_sum(y3, src_flat, dst_flat, zeros)
  imp, st = _tc_final(p3, r3, batch3d, Wimp, bimp.reshape(1, 3),
                      Wst[:_H], Wst[_H:], bst.reshape(1, 3))
  return (imp, st)
